# Initial kernel scaffold; baseline (speedup 1.0000x reference)
import jax
import jax.numpy as jnp
from jax.experimental import pallas as pl


def kernel(x, edge_index, edge_attr, params):
    raise NotImplementedError("write your pallas kernel here")


# speedup vs baseline: 36.0379x; 36.0379x over previous
import functools

import jax
import jax.numpy as jnp
from jax import lax
from jax.experimental import pallas as pl
from jax.experimental.pallas import tpu as pltpu
from jax.experimental.pallas import tpu_sc as plsc

N = 10000
E = 320000
H = 4

_C = 80
_EPT = E // 32
_NCH = _EPT // _C
_NPT = N // 16


def _ae_body(eblk, ve, ae0, ae1, ae2, esum):
    i = pl.program_id(0)
    a = jnp.dot(eblk[...], ve[...], preferred_element_type=jnp.float32)
    z12 = jnp.zeros((a.shape[0], 12), jnp.float32)
    z8 = jnp.zeros((a.shape[0], 8), jnp.float32)
    ae0[...] = jnp.concatenate([a[:, 0:4], z12], axis=1)
    ae1[...] = jnp.concatenate([a[:, 4:8], z12], axis=1)
    ae2[...] = jnp.concatenate([a[:, 8:16], z8], axis=1)

    @pl.when(i == 0)
    def _():
        esum[...] = jnp.zeros_like(esum)

    esum[...] += jnp.sum(eblk[...], axis=0, keepdims=True)


def _ae_kernel(edge_attr, ve_cat):
    be = 2000
    grid = (E // be,)
    return pl.pallas_call(
        _ae_body,
        grid=grid,
        in_specs=[
            pl.BlockSpec((be, 4), lambda i: (i, 0)),
            pl.BlockSpec((4, 16), lambda i: (0, 0)),
        ],
        out_specs=[
            pl.BlockSpec((be, 16), lambda i: (i, 0)),
            pl.BlockSpec((be, 16), lambda i: (i, 0)),
            pl.BlockSpec((be, 16), lambda i: (i, 0)),
            pl.BlockSpec((1, 4), lambda i: (0, 0)),
        ],
        out_shape=[
            jax.ShapeDtypeStruct((E, 16), jnp.float32),
            jax.ShapeDtypeStruct((E, 16), jnp.float32),
            jax.ShapeDtypeStruct((E, 16), jnp.float32),
            jax.ShapeDtypeStruct((1, 4), jnp.float32),
        ],
    )(edge_attr, ve_cat)


def _prep_body(hc, ch, hblk, wcat, aeloop, xs, asrc, adst, init, sinit):
    m = jnp.dot(hblk[...], wcat[...], preferred_element_type=jnp.float32)
    xsv = m[:, :hc]
    av = m[:, hc:hc + 4]
    dv = m[:, hc + 4:hc + 8]
    xs[...] = xsv
    asrc[...] = av
    adst[...] = dv
    al = av + dv + aeloop[...]
    exl = jnp.exp(jnp.where(al > 0, al, al * 0.2))
    half = 0.5 * exl
    sinit[...] = half
    parts = [xsv[:, h * ch:(h + 1) * ch] * half[:, h:h + 1] for h in range(H)]
    init[...] = jnp.concatenate(parts, axis=1)


def _prep_kernel(h_in, wcat, aeloop, hc, ch):
    bn = 1000
    grid = (N // bn,)
    k = wcat.shape[1]
    return pl.pallas_call(
        functools.partial(_prep_body, hc, ch),
        grid=grid,
        in_specs=[
            pl.BlockSpec((bn, 128), lambda i: (i, 0)),
            pl.BlockSpec((128, k), lambda i: (0, 0)),
            pl.BlockSpec((1, 4), lambda i: (0, 0)),
        ],
        out_specs=[
            pl.BlockSpec((bn, hc), lambda i: (i, 0)),
            pl.BlockSpec((bn, 4), lambda i: (i, 0)),
            pl.BlockSpec((bn, 4), lambda i: (i, 0)),
            pl.BlockSpec((bn, hc), lambda i: (i, 0)),
            pl.BlockSpec((bn, 4), lambda i: (i, 0)),
        ],
        out_shape=[
            jax.ShapeDtypeStruct((N, hc), jnp.float32),
            jax.ShapeDtypeStruct((N, 4), jnp.float32),
            jax.ShapeDtypeStruct((N, 4), jnp.float32),
            jax.ShapeDtypeStruct((N, hc), jnp.float32),
            jax.ShapeDtypeStruct((N, 4), jnp.float32),
        ],
    )(h_in, wcat, aeloop)


def _finish_body(acc2, s2, b, out):
    acc = acc2[0] + acc2[1]
    s = s2[0] + s2[1]
    parts = [acc[:, h * 32:(h + 1) * 32] / (s[:, h:h + 1] + 1e-16)
             for h in range(H)]
    o = jnp.concatenate(parts, axis=1) + b[...]
    out[...] = jnp.maximum(o, 0.0)


def _finish_kernel(acc2, s2, b):
    bn = 1000
    return pl.pallas_call(
        _finish_body,
        grid=(N // bn,),
        in_specs=[
            pl.BlockSpec((2, bn, 128), lambda i: (0, i, 0)),
            pl.BlockSpec((2, bn, 16), lambda i: (0, i, 0)),
            pl.BlockSpec((1, 128), lambda i: (0, 0)),
        ],
        out_specs=pl.BlockSpec((bn, 128), lambda i: (i, 0)),
        out_shape=jax.ShapeDtypeStruct((N, 128), jnp.float32),
    )(acc2, s2, b)


def _final_body(acc2, s2, bloc, bpinn, loc, pinn):
    acc = acc2[0] + acc2[1]
    s = s2[0] + s2[1]
    lsum = sum(acc[:, h * 32:(h + 1) * 32] / (s[:, h:h + 1] + 1e-16)
               for h in range(H))
    loc[...] = jnp.maximum(0.25 * lsum + bloc[...], 0.0)
    psum = sum(acc[:, 128 + h * 2:128 + (h + 1) * 2] / (s[:, 4 + h:5 + h] + 1e-16)
               for h in range(H))
    pinn[...] = 0.25 * psum + bpinn[...]


def _final_kernel(acc2, s2, bloc, bpinn):
    bn = 1000
    return pl.pallas_call(
        _final_body,
        grid=(N // bn,),
        in_specs=[
            pl.BlockSpec((2, bn, 144), lambda i: (0, i, 0)),
            pl.BlockSpec((2, bn, 16), lambda i: (0, i, 0)),
            pl.BlockSpec((1, 32), lambda i: (0, 0)),
            pl.BlockSpec((1, 2), lambda i: (0, 0)),
        ],
        out_specs=[
            pl.BlockSpec((bn, 32), lambda i: (i, 0)),
            pl.BlockSpec((bn, 2), lambda i: (i, 0)),
        ],
        out_shape=[
            jax.ShapeDtypeStruct((N, 32), jnp.float32),
            jax.ShapeDtypeStruct((N, 2), jnp.float32),
        ],
    )(acc2, s2, bloc, bpinn)


def _cls_body(locr, w, b, out):
    out[...] = jnp.dot(locr[...], w[...],
                       preferred_element_type=jnp.float32) + b[...]


def _cls_kernel(locr, w, b):
    return pl.pallas_call(
        _cls_body,
        out_shape=jax.ShapeDtypeStruct((100, 200), jnp.float32),
    )(locr, w, b)


def _make_sc_pass(r):
    mesh = plsc.VectorSubcoreMesh(core_axis_name="c", subcore_axis_name="s")

    @functools.partial(
        pl.kernel,
        out_type=[
            jax.ShapeDtypeStruct((2, N, r), jnp.float32),
            jax.ShapeDtypeStruct((2, N, 16), jnp.float32),
        ],
        mesh=mesh,
        compiler_params=pltpu.CompilerParams(needs_layout_passes=False,
                                             use_tc_tiling_on_sc=False),
        scratch_types=[
            pltpu.VMEM_SHARED((N, r), jnp.float32),
            pltpu.VMEM_SHARED((N, 16), jnp.float32),
            pltpu.VMEM((_C,), jnp.int32),
            pltpu.VMEM((_C,), jnp.int32),
            pltpu.VMEM((_C, 16), jnp.float32),
            pltpu.VMEM((_C, 16), jnp.float32),
            pltpu.VMEM((_C, 16), jnp.float32),
            pltpu.VMEM((_C, 16), jnp.float32),
            pltpu.VMEM((_C, r), jnp.float32),
            pltpu.SemaphoreType.DMA,
            pltpu.SemaphoreType.DMA,
            pltpu.SemaphoreType.DMA,
        ],
    )
    def sc_pass(src_h, dst_h, ae_h, asrc_h, adst_h, xs_h, init_h, sinit_h,
                acc_out, s_out,
                acc_sh, s_sh, srcv, dstv, aev, asg, adg, exv, xsg,
                sem1, sem2, sem3):
        cid = lax.axis_index("c")
        sid = lax.axis_index("s")

        def _row_split(fn):
            @pl.when(sid < 15)
            def _():
                r0 = pl.multiple_of(sid * 632, 8)
                fn(r0, 632)

            @pl.when(sid == 15)
            def _():
                fn(15 * 632, N - 15 * 632)

        _row_split(lambda r0, nr: (
            pltpu.sync_copy(init_h.at[pl.ds(r0, nr)],
                            acc_sh.at[pl.ds(r0, nr)]),
            pltpu.sync_copy(sinit_h.at[pl.ds(r0, nr)],
                            s_sh.at[pl.ds(r0, nr)]),
        ))
        plsc.subcore_barrier()

        tile_base = (cid * 16 + sid) * _EPT
        cidx = 4 + jnp.minimum(lax.iota(jnp.int32, 16) >> 1, 3)

        def chunk_body(i, carry):
            base = tile_base + i * _C
            pltpu.sync_copy(src_h.at[pl.ds(base, _C)], srcv)
            pltpu.sync_copy(dst_h.at[pl.ds(base, _C)], dstv)
            pltpu.sync_copy(ae_h.at[pl.ds(base, _C)], aev)
            c1 = pltpu.async_copy(asrc_h.at[srcv], asg, sem1)
            c2 = pltpu.async_copy(adst_h.at[dstv], adg, sem2)
            c3 = pltpu.async_copy(xs_h.at[srcv], xsg, sem3)
            c1.wait()
            c2.wait()
            c3.wait()

            def edge_body(e, carry2):
                al = asg[e, :] + adg[e, :] + aev[e, :]
                ex = jnp.exp(jnp.where(al > 0, al, al * 0.2))
                exv[e, :] = ex
                erow = jnp.full((16,), e, jnp.int32)
                for h in range(H):
                    w = plsc.load_gather(
                        exv, [erow, jnp.full((16,), h, jnp.int32)])
                    xsg[e, pl.ds(h * 32, 16)] = xsg[e, pl.ds(h * 32, 16)] * w
                    xsg[e, pl.ds(h * 32 + 16, 16)] = (
                        xsg[e, pl.ds(h * 32 + 16, 16)] * w)
                if r == 144:
                    w = plsc.load_gather(exv, [erow, cidx])
                    xsg[e, pl.ds(128, 16)] = xsg[e, pl.ds(128, 16)] * w
                return carry2

            lax.fori_loop(0, _C, edge_body, 0)
            pltpu.sync_copy(exv, s_sh.at[dstv], add=True)
            pltpu.sync_copy(xsg, acc_sh.at[dstv], add=True)
            return carry

        lax.fori_loop(0, _NCH, chunk_body, 0)
        plsc.subcore_barrier()
        _row_split(lambda r0, nr: (
            pltpu.sync_copy(acc_sh.at[pl.ds(r0, nr)],
                            acc_out.at[cid, pl.ds(r0, nr)]),
            pltpu.sync_copy(s_sh.at[pl.ds(r0, nr)],
                            s_out.at[cid, pl.ds(r0, nr)]),
        ))

    return sc_pass


_sc_pass_128 = _make_sc_pass(128)
_sc_pass_144 = _make_sc_pass(144)


def _fold(p):
    w = p['W']
    d_in = w.shape[0]
    heads, ch = p['a_src'].shape
    wr = w.reshape(d_in, heads, ch)
    u_src = jnp.einsum('dhc,hc->dh', wr, p['a_src'])
    u_dst = jnp.einsum('dhc,hc->dh', wr, p['a_dst'])
    v_e = jnp.einsum('dhc,hc->dh', p['W_e'].reshape(4, heads, ch), p['a_e'])
    wcat = jnp.concatenate([w, u_src, u_dst], axis=1)
    return wcat, v_e


def _pad16(a4, b4=None):
    z = jnp.zeros((N, 8), jnp.float32)
    if b4 is None:
        return jnp.concatenate([a4, jnp.zeros((N, 4), jnp.float32), z], axis=1)
    return jnp.concatenate([a4, b4, z], axis=1)


def kernel(x, edge_index, edge_attr, params):
    src = edge_index[0]
    dst = edge_index[1]

    g0, g1 = params['gnns']
    ploc = params['localize']
    ppinn = params['pinn']
    wcat0, ve0 = _fold(g0)
    wcat1, ve1 = _fold(g1)
    wcatl, vel = _fold(ploc)
    wcatp, vep = _fold(ppinn)

    ve_cat = jnp.concatenate([ve0, ve1, vel, vep], axis=1)
    ae0, ae1, ae2, esum = _ae_kernel(edge_attr, ve_cat)
    mean_e = esum / E
    al0 = mean_e @ ve0
    al1 = mean_e @ ve1
    all_ = mean_e @ vel
    alp = mean_e @ vep

    xs, a_s, a_d, init, sinit = _prep_kernel(x, wcat0, al0, 128, 32)
    acc2, s2 = _sc_pass_128(src, dst, ae0, _pad16(a_s), _pad16(a_d),
                            xs, init, _pad16(sinit))
    h1 = _finish_kernel(acc2, s2, g0['b'].reshape(1, 128))

    xs, a_s, a_d, init, sinit = _prep_kernel(h1, wcat1, al1, 128, 32)
    acc2, s2 = _sc_pass_128(src, dst, ae1, _pad16(a_s), _pad16(a_d),
                            xs, init, _pad16(sinit))
    h2 = _finish_kernel(acc2, s2, g1['b'].reshape(1, 128))

    xsl, asl, adl, initl, sinitl = _prep_kernel(h2, wcatl, all_, 128, 32)
    xsp, asp, adp, initp, sinitp = _prep_kernel(h2, wcatp, alp, 8, 2)
    z8 = jnp.zeros((N, 8), jnp.float32)
    xs_cat = jnp.concatenate([xsl, xsp, z8], axis=1)
    init_cat = jnp.concatenate([initl, initp, z8], axis=1)
    acc2, s2 = _sc_pass_144(src, dst, ae2, _pad16(asl, asp), _pad16(adl, adp),
                            xs_cat, init_cat, _pad16(sinitl, sinitp))
    loc, pinn = _final_kernel(acc2, s2, ploc['b'].reshape(1, 32),
                              ppinn['b'].reshape(1, 2))

    locr = loc.reshape(100, 3200)
    out2 = _cls_kernel(locr, params['cls_W'],
                       params['cls_b'].reshape(1, 200))
    return pinn, out2


# speedup vs baseline: 46.8007x; 1.2987x over previous
import functools

import jax
import jax.numpy as jnp
from jax import lax
from jax.experimental import pallas as pl
from jax.experimental.pallas import tpu as pltpu
from jax.experimental.pallas import tpu_sc as plsc

N = 10000
E = 320000
H = 4

_C = 40
_EPT = E // 32
_NCH = _EPT // _C
_NPT = N // 16


def _ae_body(eblk, ve, ae0, ae1, ae2, esum):
    i = pl.program_id(0)
    a = jnp.dot(eblk[...], ve[...], preferred_element_type=jnp.float32)
    z12 = jnp.zeros((a.shape[0], 12), jnp.float32)
    z8 = jnp.zeros((a.shape[0], 8), jnp.float32)
    ae0[...] = jnp.concatenate([a[:, 0:4], z12], axis=1)
    ae1[...] = jnp.concatenate([a[:, 4:8], z12], axis=1)
    ae2[...] = jnp.concatenate([a[:, 8:16], z8], axis=1)

    @pl.when(i == 0)
    def _():
        esum[...] = jnp.zeros_like(esum)

    esum[...] += jnp.sum(eblk[...], axis=0, keepdims=True)


def _ae_kernel(edge_attr, ve_cat):
    be = 2000
    grid = (E // be,)
    return pl.pallas_call(
        _ae_body,
        grid=grid,
        in_specs=[
            pl.BlockSpec((be, 4), lambda i: (i, 0)),
            pl.BlockSpec((4, 16), lambda i: (0, 0)),
        ],
        out_specs=[
            pl.BlockSpec((be, 16), lambda i: (i, 0)),
            pl.BlockSpec((be, 16), lambda i: (i, 0)),
            pl.BlockSpec((be, 16), lambda i: (i, 0)),
            pl.BlockSpec((1, 4), lambda i: (0, 0)),
        ],
        out_shape=[
            jax.ShapeDtypeStruct((E, 16), jnp.float32),
            jax.ShapeDtypeStruct((E, 16), jnp.float32),
            jax.ShapeDtypeStruct((E, 16), jnp.float32),
            jax.ShapeDtypeStruct((1, 4), jnp.float32),
        ],
    )(edge_attr, ve_cat)


def _prep_body(hc, ch, hblk, wcat, aeloop, xs, asrc, adst, init, sinit):
    m = jnp.dot(hblk[...], wcat[...], preferred_element_type=jnp.float32)
    xsv = m[:, :hc]
    av = m[:, hc:hc + 4]
    dv = m[:, hc + 4:hc + 8]
    xs[...] = xsv
    asrc[...] = av
    adst[...] = dv
    al = av + dv + aeloop[...]
    exl = jnp.exp(jnp.where(al > 0, al, al * 0.2))
    half = 0.5 * exl
    sinit[...] = half
    parts = [xsv[:, h * ch:(h + 1) * ch] * half[:, h:h + 1] for h in range(H)]
    init[...] = jnp.concatenate(parts, axis=1)


def _prep_kernel(h_in, wcat, aeloop, hc, ch):
    bn = 1000
    grid = (N // bn,)
    k = wcat.shape[1]
    return pl.pallas_call(
        functools.partial(_prep_body, hc, ch),
        grid=grid,
        in_specs=[
            pl.BlockSpec((bn, 128), lambda i: (i, 0)),
            pl.BlockSpec((128, k), lambda i: (0, 0)),
            pl.BlockSpec((1, 4), lambda i: (0, 0)),
        ],
        out_specs=[
            pl.BlockSpec((bn, hc), lambda i: (i, 0)),
            pl.BlockSpec((bn, 4), lambda i: (i, 0)),
            pl.BlockSpec((bn, 4), lambda i: (i, 0)),
            pl.BlockSpec((bn, hc), lambda i: (i, 0)),
            pl.BlockSpec((bn, 4), lambda i: (i, 0)),
        ],
        out_shape=[
            jax.ShapeDtypeStruct((N, hc), jnp.float32),
            jax.ShapeDtypeStruct((N, 4), jnp.float32),
            jax.ShapeDtypeStruct((N, 4), jnp.float32),
            jax.ShapeDtypeStruct((N, hc), jnp.float32),
            jax.ShapeDtypeStruct((N, 4), jnp.float32),
        ],
    )(h_in, wcat, aeloop)


def _finish_body(acc2, s2, b, out):
    acc = acc2[0] + acc2[1]
    s = s2[0] + s2[1]
    parts = [acc[:, h * 32:(h + 1) * 32] / (s[:, h:h + 1] + 1e-16)
             for h in range(H)]
    o = jnp.concatenate(parts, axis=1) + b[...]
    out[...] = jnp.maximum(o, 0.0)


def _finish_kernel(acc2, s2, b):
    bn = 1000
    return pl.pallas_call(
        _finish_body,
        grid=(N // bn,),
        in_specs=[
            pl.BlockSpec((2, bn, 128), lambda i: (0, i, 0)),
            pl.BlockSpec((2, bn, 16), lambda i: (0, i, 0)),
            pl.BlockSpec((1, 128), lambda i: (0, 0)),
        ],
        out_specs=pl.BlockSpec((bn, 128), lambda i: (i, 0)),
        out_shape=jax.ShapeDtypeStruct((N, 128), jnp.float32),
    )(acc2, s2, b)


def _final_body(acc2, s2, bloc, bpinn, loc, pinn):
    acc = acc2[0] + acc2[1]
    s = s2[0] + s2[1]
    lsum = sum(acc[:, h * 32:(h + 1) * 32] / (s[:, h:h + 1] + 1e-16)
               for h in range(H))
    loc[...] = jnp.maximum(0.25 * lsum + bloc[...], 0.0)
    psum = sum(acc[:, 128 + h * 2:128 + (h + 1) * 2] / (s[:, 4 + h:5 + h] + 1e-16)
               for h in range(H))
    pinn[...] = 0.25 * psum + bpinn[...]


def _final_kernel(acc2, s2, bloc, bpinn):
    bn = 1000
    return pl.pallas_call(
        _final_body,
        grid=(N // bn,),
        in_specs=[
            pl.BlockSpec((2, bn, 144), lambda i: (0, i, 0)),
            pl.BlockSpec((2, bn, 16), lambda i: (0, i, 0)),
            pl.BlockSpec((1, 32), lambda i: (0, 0)),
            pl.BlockSpec((1, 2), lambda i: (0, 0)),
        ],
        out_specs=[
            pl.BlockSpec((bn, 32), lambda i: (i, 0)),
            pl.BlockSpec((bn, 2), lambda i: (i, 0)),
        ],
        out_shape=[
            jax.ShapeDtypeStruct((N, 32), jnp.float32),
            jax.ShapeDtypeStruct((N, 2), jnp.float32),
        ],
    )(acc2, s2, bloc, bpinn)


def _cls_body(locr, w, b, out):
    out[...] = jnp.dot(locr[...], w[...],
                       preferred_element_type=jnp.float32) + b[...]


def _cls_kernel(locr, w, b):
    return pl.pallas_call(
        _cls_body,
        out_shape=jax.ShapeDtypeStruct((100, 200), jnp.float32),
    )(locr, w, b)


def _make_sc_pass(r):
    mesh = plsc.VectorSubcoreMesh(core_axis_name="c", subcore_axis_name="s")

    buf_types = []
    for _ in range(3):
        buf_types += [
            pltpu.VMEM((_C,), jnp.int32),
            pltpu.VMEM((_C,), jnp.int32),
            pltpu.VMEM((_C, 16), jnp.float32),
            pltpu.VMEM((_C, 16), jnp.float32),
            pltpu.VMEM((_C, 16), jnp.float32),
            pltpu.VMEM((_C, 16), jnp.float32),
            pltpu.VMEM((_C, r), jnp.float32),
        ]
    sem_types = [pltpu.SemaphoreType.DMA] * 18

    @functools.partial(
        pl.kernel,
        out_type=[
            jax.ShapeDtypeStruct((2, N, r), jnp.float32),
            jax.ShapeDtypeStruct((2, N, 16), jnp.float32),
        ],
        mesh=mesh,
        compiler_params=pltpu.CompilerParams(needs_layout_passes=False,
                                             use_tc_tiling_on_sc=False),
        scratch_types=[
            pltpu.VMEM_SHARED((N, r), jnp.float32),
            pltpu.VMEM_SHARED((N, 16), jnp.float32),
        ] + buf_types + sem_types,
    )
    def sc_pass(src_h, dst_h, ae_h, asrc_h, adst_h, xs_h, init_h, sinit_h,
                acc_out, s_out, acc_sh, s_sh, *scratch):
        bufs = [scratch[7 * k:7 * (k + 1)] for k in range(3)]
        sems = [scratch[21 + 6 * k:21 + 6 * (k + 1)] for k in range(3)]
        cid = lax.axis_index("c")
        sid = lax.axis_index("s")

        def _row_split(fn):
            @pl.when(sid < 15)
            def _():
                r0 = pl.multiple_of(sid * 632, 8)
                fn(r0, 632)

            @pl.when(sid == 15)
            def _():
                fn(15 * 632, N - 15 * 632)

        _row_split(lambda r0, nr: (
            pltpu.sync_copy(init_h.at[pl.ds(r0, nr)],
                            acc_sh.at[pl.ds(r0, nr)]),
            pltpu.sync_copy(sinit_h.at[pl.ds(r0, nr)],
                            s_sh.at[pl.ds(r0, nr)]),
        ))
        plsc.subcore_barrier()

        tile_base = (cid * 16 + sid) * _EPT
        cidx = 4 + jnp.minimum(lax.iota(jnp.int32, 16) >> 1, 3)

        def fire_idx(k, base):
            srcv, dstv, aev = bufs[k][0], bufs[k][1], bufs[k][2]
            isem = sems[k][0]
            pltpu.async_copy(src_h.at[pl.ds(base, _C)], srcv, isem)
            pltpu.async_copy(dst_h.at[pl.ds(base, _C)], dstv, isem)
            pltpu.async_copy(ae_h.at[pl.ds(base, _C)], aev, isem)

        def wait_idx(k, base):
            srcv, dstv, aev = bufs[k][0], bufs[k][1], bufs[k][2]
            isem = sems[k][0]
            pltpu.make_async_copy(src_h.at[pl.ds(base, _C)], srcv, isem).wait()
            pltpu.make_async_copy(dst_h.at[pl.ds(base, _C)], dstv, isem).wait()
            pltpu.make_async_copy(ae_h.at[pl.ds(base, _C)], aev, isem).wait()

        def fire_gather(k):
            srcv, dstv, _, asg, adg, _, xsg = bufs[k]
            pltpu.async_copy(asrc_h.at[srcv], asg, sems[k][1])
            pltpu.async_copy(adst_h.at[dstv], adg, sems[k][2])
            pltpu.async_copy(xs_h.at[srcv], xsg, sems[k][3])

        def wait_gather(k):
            srcv, dstv, _, asg, adg, _, xsg = bufs[k]
            pltpu.make_async_copy(asrc_h.at[srcv], asg, sems[k][1]).wait()
            pltpu.make_async_copy(adst_h.at[dstv], adg, sems[k][2]).wait()
            pltpu.make_async_copy(xs_h.at[srcv], xsg, sems[k][3]).wait()

        def fire_scatter(k):
            _, dstv, _, _, _, exv, xsg = bufs[k]
            pltpu.async_copy(exv, s_sh.at[dstv], sems[k][4], add=True)
            pltpu.async_copy(xsg, acc_sh.at[dstv], sems[k][5], add=True)

        def wait_scatter(k):
            _, dstv, _, _, _, exv, xsg = bufs[k]
            pltpu.make_async_copy(exv, s_sh.at[dstv], sems[k][4]).wait()
            pltpu.make_async_copy(xsg, acc_sh.at[dstv], sems[k][5]).wait()

        def compute(k):
            _, _, aev, asg, adg, exv, xsg = bufs[k]

            def edge_body(e, carry2):
                al = asg[e, :] + adg[e, :] + aev[e, :]
                ex = jnp.exp(jnp.where(al > 0, al, al * 0.2))
                exv[e, :] = ex
                erow = jnp.full((16,), e, jnp.int32)
                for h in range(H):
                    w = plsc.load_gather(
                        exv, [erow, jnp.full((16,), h, jnp.int32)])
                    xsg[e, pl.ds(h * 32, 16)] = xsg[e, pl.ds(h * 32, 16)] * w
                    xsg[e, pl.ds(h * 32 + 16, 16)] = (
                        xsg[e, pl.ds(h * 32 + 16, 16)] * w)
                if r == 144:
                    w = plsc.load_gather(exv, [erow, cidx])
                    xsg[e, pl.ds(128, 16)] = xsg[e, pl.ds(128, 16)] * w
                return carry2

            lax.fori_loop(0, _C, edge_body, 0)

        fire_idx(0, tile_base)
        wait_idx(0, tile_base)
        fire_gather(0)

        def chunk_body(i, carry):
            for s in range(3):
                sn = (s + 1) % 3

                @pl.when(i % 3 == s)
                def _(s=s, sn=sn):
                    nbase = tile_base + (i + 1) * _C

                    @pl.when(i >= 2)
                    def _():
                        wait_scatter(sn)

                    @pl.when(i + 1 < _NCH)
                    def _():
                        fire_idx(sn, nbase)
                        wait_idx(sn, nbase)
                        fire_gather(sn)

                    wait_gather(s)
                    compute(s)
                    fire_scatter(s)

            return carry

        lax.fori_loop(0, _NCH, chunk_body, 0)
        wait_scatter((_NCH - 2) % 3)
        wait_scatter((_NCH - 1) % 3)
        plsc.subcore_barrier()
        _row_split(lambda r0, nr: (
            pltpu.sync_copy(acc_sh.at[pl.ds(r0, nr)],
                            acc_out.at[cid, pl.ds(r0, nr)]),
            pltpu.sync_copy(s_sh.at[pl.ds(r0, nr)],
                            s_out.at[cid, pl.ds(r0, nr)]),
        ))

    return sc_pass


_sc_pass_128 = _make_sc_pass(128)
_sc_pass_144 = _make_sc_pass(144)


def _fold(p):
    w = p['W']
    d_in = w.shape[0]
    heads, ch = p['a_src'].shape
    wr = w.reshape(d_in, heads, ch)
    u_src = jnp.einsum('dhc,hc->dh', wr, p['a_src'])
    u_dst = jnp.einsum('dhc,hc->dh', wr, p['a_dst'])
    v_e = jnp.einsum('dhc,hc->dh', p['W_e'].reshape(4, heads, ch), p['a_e'])
    wcat = jnp.concatenate([w, u_src, u_dst], axis=1)
    return wcat, v_e


def _pad16(a4, b4=None):
    z = jnp.zeros((N, 8), jnp.float32)
    if b4 is None:
        return jnp.concatenate([a4, jnp.zeros((N, 4), jnp.float32), z], axis=1)
    return jnp.concatenate([a4, b4, z], axis=1)


def kernel(x, edge_index, edge_attr, params):
    src = edge_index[0]
    dst = edge_index[1]

    g0, g1 = params['gnns']
    ploc = params['localize']
    ppinn = params['pinn']
    wcat0, ve0 = _fold(g0)
    wcat1, ve1 = _fold(g1)
    wcatl, vel = _fold(ploc)
    wcatp, vep = _fold(ppinn)

    ve_cat = jnp.concatenate([ve0, ve1, vel, vep], axis=1)
    ae0, ae1, ae2, esum = _ae_kernel(edge_attr, ve_cat)
    mean_e = esum / E
    al0 = mean_e @ ve0
    al1 = mean_e @ ve1
    all_ = mean_e @ vel
    alp = mean_e @ vep

    xs, a_s, a_d, init, sinit = _prep_kernel(x, wcat0, al0, 128, 32)
    acc2, s2 = _sc_pass_128(src, dst, ae0, _pad16(a_s), _pad16(a_d),
                            xs, init, _pad16(sinit))
    h1 = _finish_kernel(acc2, s2, g0['b'].reshape(1, 128))

    xs, a_s, a_d, init, sinit = _prep_kernel(h1, wcat1, al1, 128, 32)
    acc2, s2 = _sc_pass_128(src, dst, ae1, _pad16(a_s), _pad16(a_d),
                            xs, init, _pad16(sinit))
    h2 = _finish_kernel(acc2, s2, g1['b'].reshape(1, 128))

    xsl, asl, adl, initl, sinitl = _prep_kernel(h2, wcatl, all_, 128, 32)
    xsp, asp, adp, initp, sinitp = _prep_kernel(h2, wcatp, alp, 8, 2)
    z8 = jnp.zeros((N, 8), jnp.float32)
    xs_cat = jnp.concatenate([xsl, xsp, z8], axis=1)
    init_cat = jnp.concatenate([initl, initp, z8], axis=1)
    acc2, s2 = _sc_pass_144(src, dst, ae2, _pad16(asl, asp), _pad16(adl, adp),
                            xs_cat, init_cat, _pad16(sinitl, sinitp))
    loc, pinn = _final_kernel(acc2, s2, ploc['b'].reshape(1, 32),
                              ppinn['b'].reshape(1, 2))

    locr = loc.reshape(100, 3200)
    out2 = _cls_kernel(locr, params['cls_W'],
                       params['cls_b'].reshape(1, 200))
    return pinn, out2
